# Initial kernel scaffold; baseline (speedup 1.0000x reference)
import jax
import jax.numpy as jnp
from jax.experimental import pallas as pl


def kernel(hidden_states, w_router, w_gate, w_up, w_down):
    raise NotImplementedError("write your pallas kernel here")


# speedup vs baseline: 1.4561x; 1.4561x over previous
import functools
import math

import jax
import jax.numpy as jnp
from jax.experimental import pallas as pl
from jax.experimental.pallas import tpu as pltpu


def _router_body(num_experts, capacity, x_ref, wr_ref, mc_ref, mr_ref, pc_ref,
                 aux_ref):
    n = x_ref.shape[0]
    e_dim = num_experts
    x = x_ref[...]
    wr = wr_ref[...]
    logits = jnp.dot(x, wr, preferred_element_type=jnp.float32)
    mx = jnp.max(logits, axis=-1, keepdims=True)
    ex = jnp.exp(logits - mx)
    probs = ex / jnp.sum(ex, axis=-1, keepdims=True)

    iota_e = jax.lax.broadcasted_iota(jnp.int32, (n, e_dim), 1)
    m1 = jnp.max(probs, axis=-1, keepdims=True)
    e1 = jnp.min(jnp.where(probs == m1, iota_e, e_dim), axis=-1, keepdims=True)
    probs2 = jnp.where(iota_e == e1, -1.0, probs)
    m2 = jnp.max(probs2, axis=-1, keepdims=True)
    e2 = jnp.min(jnp.where(probs2 == m2, iota_e, e_dim), axis=-1, keepdims=True)
    s = m1 + m2
    p1 = m1 / s
    p2 = m2 / s

    imp = jnp.sum(probs, axis=0, keepdims=True)
    load = jnp.sum((iota_e == e1).astype(jnp.float32)
                   + (iota_e == e2).astype(jnp.float32), axis=0, keepdims=True)
    aux_ref[...] = (e_dim * jnp.sum(imp * load, keepdims=True)) / (n * 2)

    e1f = e1.astype(jnp.float32)
    e2f = e2.astype(jnp.float32)
    zeros4 = jnp.zeros((n, 4), jnp.float32)
    packed = jnp.concatenate([e1f, e2f, p1, p2, zeros4], axis=1)
    packed_t = packed.T
    e1r = packed_t[0:1, :].astype(jnp.int32)
    e2r = packed_t[1:2, :].astype(jnp.int32)
    p1r = packed_t[2:3, :]
    p2r = packed_t[3:4, :]

    ch = 512
    tok_row = jax.lax.broadcasted_iota(jnp.int32, (1, n), 1)
    r1_parts = []
    r2_parts = []
    for c in range(n // ch):
        sl = slice(c * ch, (c + 1) * ch)
        tcol = jax.lax.broadcasted_iota(jnp.int32, (ch, 1), 0) + c * ch
        for k, (ei, pi) in enumerate(((e1[sl], p1[sl]), (e2[sl], p2[sl]))):
            cnt = jnp.zeros((ch, 1), jnp.int32)
            for kj, (ejr, pjr) in enumerate(((e1r, p1r), (e2r, p2r))):
                same = ejr == ei
                if kj < k:
                    earlier = tok_row <= tcol
                else:
                    earlier = tok_row < tcol
                beat = (pjr > pi) | ((pjr == pi) & earlier)
                cnt = cnt + jnp.sum((same & beat).astype(jnp.int32), axis=1,
                                    keepdims=True)
            if k == 0:
                r1_parts.append(cnt)
            else:
                r2_parts.append(cnt)
    r1 = jnp.concatenate(r1_parts, axis=0)
    r2 = jnp.concatenate(r2_parts, axis=0)

    zeros4i = jnp.zeros((n, 4), jnp.int32)
    mc_ref[...] = jnp.concatenate([e1, e2, r1, r2, zeros4i], axis=1)
    packed2 = jnp.concatenate([e1f, e2f, r1.astype(jnp.float32),
                               r2.astype(jnp.float32), zeros4], axis=1)
    mr_ref[...] = packed2.T.astype(jnp.int32)
    pc_ref[...] = jnp.concatenate([p1, p2, zeros4, jnp.zeros((n, 2),
                                                             jnp.float32)],
                                  axis=1)


def _ffn_body(capacity, nt, x_ref, wg_ref, wu_ref, wd_ref, mr_ref, d_ref,
              g_ref, seg_ref):
    e = pl.program_id(0)
    f = pl.program_id(1)
    n = x_ref.shape[0]
    ch = 512

    @pl.when(f == 0)
    def _build_seg():
        mr = mr_ref[...]
        for c in range(n // ch):
            sl = slice(c * ch, (c + 1) * ch)
            e1r = mr[0:1, sl]
            e2r = mr[1:2, sl]
            r1r = mr[2:3, sl]
            r2r = mr[3:4, sl]
            iota_r = jax.lax.broadcasted_iota(jnp.int32, (capacity, ch), 0)
            g_ref[:, sl] = (
                ((e1r == e) & (iota_r == r1r)).astype(jnp.bfloat16)
                + ((e2r == e) & (iota_r == r2r)).astype(jnp.bfloat16))
        seg32 = jnp.dot(g_ref[...], x_ref[...],
                        preferred_element_type=jnp.float32)
        seg_ref[...] = seg32.astype(jnp.bfloat16)

    gate = jnp.dot(seg_ref[...], wg_ref[0].astype(jnp.bfloat16),
                   preferred_element_type=jnp.float32)
    up = jnp.dot(seg_ref[...], wu_ref[0].astype(jnp.bfloat16),
                 preferred_element_type=jnp.float32)
    h = (gate * jax.nn.sigmoid(gate)) * up
    dpart = jnp.dot(h.astype(jnp.bfloat16), wd_ref[0].astype(jnp.bfloat16),
                    preferred_element_type=jnp.float32)

    @pl.when(f == 0)
    def _init_d():
        d_ref[0] = dpart

    @pl.when(f > 0)
    def _acc_d():
        d_ref[0] = d_ref[0] + dpart


def _combine_body(capacity, mc_ref, pc_ref, d_ref, out_ref, w_ref):
    e = pl.program_id(0)
    n = out_ref.shape[0]
    ch = 512
    mc = mc_ref[...]
    pc = pc_ref[...]
    iota_rw = jax.lax.broadcasted_iota(jnp.int32, (ch, capacity), 1)
    for c in range(n // ch):
        sl = slice(c * ch, (c + 1) * ch)
        e1c = mc[sl, 0:1]
        e2c = mc[sl, 1:2]
        r1c = mc[sl, 2:3]
        r2c = mc[sl, 3:4]
        p1c = pc[sl, 0:1]
        p2c = pc[sl, 1:2]
        w = (jnp.where((e1c == e) & (iota_rw == r1c), p1c, 0.0)
             + jnp.where((e2c == e) & (iota_rw == r2c), p2c, 0.0))
        w_ref[sl, :] = w.astype(jnp.bfloat16)
    dbf = d_ref[0].astype(jnp.bfloat16)
    for c in range(n // ch):
        sl = slice(c * ch, (c + 1) * ch)
        contrib = jnp.dot(w_ref[sl, :], dbf, preferred_element_type=jnp.float32)

        @pl.when(e == 0)
        def _():
            out_ref[sl, :] = contrib

        @pl.when(e > 0)
        def _():
            out_ref[sl, :] = out_ref[sl, :] + contrib


def kernel(hidden_states, w_router, w_gate, w_up, w_down):
    n, hidden = hidden_states.shape
    num_experts = w_router.shape[1]
    ffn = w_gate.shape[2]
    capacity = max(4, math.ceil(1.25 * n * 2 / num_experts))
    capacity = min(capacity, n * 2)

    mc, mr, pc, aux = pl.pallas_call(
        functools.partial(_router_body, num_experts, capacity),
        in_specs=[
            pl.BlockSpec((n, hidden), lambda: (0, 0)),
            pl.BlockSpec((hidden, num_experts), lambda: (0, 0)),
        ],
        out_specs=[
            pl.BlockSpec((n, 8), lambda: (0, 0)),
            pl.BlockSpec((8, n), lambda: (0, 0)),
            pl.BlockSpec((n, 8), lambda: (0, 0)),
            pl.BlockSpec((1, 1), lambda: (0, 0)),
        ],
        out_shape=[
            jax.ShapeDtypeStruct((n, 8), jnp.int32),
            jax.ShapeDtypeStruct((8, n), jnp.int32),
            jax.ShapeDtypeStruct((n, 8), jnp.float32),
            jax.ShapeDtypeStruct((1, 1), jnp.float32),
        ],
    )(hidden_states, w_router)

    x_bf = hidden_states.astype(jnp.bfloat16)

    tile = 512
    nt = ffn // tile
    d_all = pl.pallas_call(
        functools.partial(_ffn_body, capacity, nt),
        grid=(num_experts, nt),
        in_specs=[
            pl.BlockSpec((n, hidden), lambda e, f: (0, 0)),
            pl.BlockSpec((1, hidden, tile), lambda e, f: (e, 0, f)),
            pl.BlockSpec((1, hidden, tile), lambda e, f: (e, 0, f)),
            pl.BlockSpec((1, tile, hidden), lambda e, f: (e, f, 0)),
            pl.BlockSpec((8, n), lambda e, f: (0, 0)),
        ],
        out_specs=pl.BlockSpec((1, capacity, hidden), lambda e, f: (e, 0, 0)),
        out_shape=jax.ShapeDtypeStruct((num_experts, capacity, hidden),
                                       jnp.float32),
        scratch_shapes=[
            pltpu.VMEM((capacity, n), jnp.bfloat16),
            pltpu.VMEM((capacity, hidden), jnp.bfloat16),
        ],
    )(x_bf, w_gate, w_up, w_down, mr)

    out = pl.pallas_call(
        functools.partial(_combine_body, capacity),
        grid=(num_experts,),
        in_specs=[
            pl.BlockSpec((n, 8), lambda e: (0, 0)),
            pl.BlockSpec((n, 8), lambda e: (0, 0)),
            pl.BlockSpec((1, capacity, hidden), lambda e: (e, 0, 0)),
        ],
        out_specs=pl.BlockSpec((n, hidden), lambda e: (0, 0)),
        out_shape=jax.ShapeDtypeStruct((n, hidden), jnp.float32),
        scratch_shapes=[
            pltpu.VMEM((n, capacity), jnp.bfloat16),
        ],
    )(mc, pc, d_all)

    return out, aux[0, 0]


# speedup vs baseline: 1.4877x; 1.0217x over previous
import functools
import math

import jax
import jax.numpy as jnp
from jax.experimental import pallas as pl
from jax.experimental.pallas import tpu as pltpu


def _router_body(num_experts, capacity, x_ref, wr_ref, mc_ref, mr_ref, pc_ref,
                 aux_ref):
    n = x_ref.shape[0]
    e_dim = num_experts
    x = x_ref[...]
    wr = wr_ref[...]
    logits = jnp.dot(x, wr, preferred_element_type=jnp.float32)
    mx = jnp.max(logits, axis=-1, keepdims=True)
    ex = jnp.exp(logits - mx)
    probs = ex / jnp.sum(ex, axis=-1, keepdims=True)

    iota_e = jax.lax.broadcasted_iota(jnp.int32, (n, e_dim), 1)
    m1 = jnp.max(probs, axis=-1, keepdims=True)
    e1 = jnp.min(jnp.where(probs == m1, iota_e, e_dim), axis=-1, keepdims=True)
    probs2 = jnp.where(iota_e == e1, -1.0, probs)
    m2 = jnp.max(probs2, axis=-1, keepdims=True)
    e2 = jnp.min(jnp.where(probs2 == m2, iota_e, e_dim), axis=-1, keepdims=True)
    s = m1 + m2
    p1 = m1 / s
    p2 = m2 / s

    imp = jnp.sum(probs, axis=0, keepdims=True)
    load = jnp.sum((iota_e == e1).astype(jnp.float32)
                   + (iota_e == e2).astype(jnp.float32), axis=0, keepdims=True)
    aux_ref[...] = (e_dim * jnp.sum(imp * load, keepdims=True)) / (n * 2)

    e1f = e1.astype(jnp.float32)
    e2f = e2.astype(jnp.float32)
    zeros4 = jnp.zeros((n, 4), jnp.float32)
    packed = jnp.concatenate([e1f, e2f, p1, p2, zeros4], axis=1)
    packed_t = packed.T
    e1r = packed_t[0:1, :].astype(jnp.int32)
    e2r = packed_t[1:2, :].astype(jnp.int32)
    p1r = packed_t[2:3, :]
    p2r = packed_t[3:4, :]

    ch = 512
    tok_row = jax.lax.broadcasted_iota(jnp.int32, (1, n), 1)
    r1_parts = []
    r2_parts = []
    for c in range(n // ch):
        sl = slice(c * ch, (c + 1) * ch)
        tcol = jax.lax.broadcasted_iota(jnp.int32, (ch, 1), 0) + c * ch
        for k, (ei, pi) in enumerate(((e1[sl], p1[sl]), (e2[sl], p2[sl]))):
            cnt = jnp.zeros((ch, 1), jnp.int32)
            for kj, (ejr, pjr) in enumerate(((e1r, p1r), (e2r, p2r))):
                same = ejr == ei
                if kj < k:
                    earlier = tok_row <= tcol
                else:
                    earlier = tok_row < tcol
                beat = (pjr > pi) | ((pjr == pi) & earlier)
                cnt = cnt + jnp.sum((same & beat).astype(jnp.int32), axis=1,
                                    keepdims=True)
            if k == 0:
                r1_parts.append(cnt)
            else:
                r2_parts.append(cnt)
    r1 = jnp.concatenate(r1_parts, axis=0)
    r2 = jnp.concatenate(r2_parts, axis=0)

    zeros4i = jnp.zeros((n, 4), jnp.int32)
    mc_ref[...] = jnp.concatenate([e1, e2, r1, r2, zeros4i], axis=1)
    packed2 = jnp.concatenate([e1f, e2f, r1.astype(jnp.float32),
                               r2.astype(jnp.float32), zeros4], axis=1)
    mr_ref[...] = packed2.T.astype(jnp.int32)
    pc_ref[...] = jnp.concatenate([p1, p2, zeros4, jnp.zeros((n, 2),
                                                             jnp.float32)],
                                  axis=1)


def _ffn_body(capacity, nt, x_ref, wg_ref, wu_ref, wd_ref, mr_ref, d_ref,
              g_ref, seg_ref):
    e = pl.program_id(0)
    f = pl.program_id(1)
    n = x_ref.shape[0]
    ch = 512

    @pl.when(f == 0)
    def _build_seg():
        mr = mr_ref[...]
        for c in range(n // ch):
            sl = slice(c * ch, (c + 1) * ch)
            e1r = mr[0:1, sl]
            e2r = mr[1:2, sl]
            r1r = mr[2:3, sl]
            r2r = mr[3:4, sl]
            iota_r = jax.lax.broadcasted_iota(jnp.int32, (capacity, ch), 0)
            g_ref[:, sl] = (
                ((e1r == e) & (iota_r == r1r)).astype(jnp.bfloat16)
                + ((e2r == e) & (iota_r == r2r)).astype(jnp.bfloat16))
        seg32 = jnp.dot(g_ref[...], x_ref[...],
                        preferred_element_type=jnp.float32)
        seg_ref[...] = seg32.astype(jnp.bfloat16)

    gate = jnp.dot(seg_ref[...], wg_ref[0].astype(jnp.bfloat16),
                   preferred_element_type=jnp.float32)
    up = jnp.dot(seg_ref[...], wu_ref[0].astype(jnp.bfloat16),
                 preferred_element_type=jnp.float32)
    h = (gate * jax.nn.sigmoid(gate)) * up
    dpart = jnp.dot(h.astype(jnp.bfloat16), wd_ref[0].astype(jnp.bfloat16),
                    preferred_element_type=jnp.float32)

    @pl.when(f == 0)
    def _init_d():
        d_ref[0] = dpart.astype(jnp.bfloat16)

    @pl.when(f > 0)
    def _acc_d():
        d_ref[0] = (d_ref[0].astype(jnp.float32) + dpart).astype(jnp.bfloat16)


def _combine_body(capacity, mc_ref, pc_ref, d_ref, out_ref, w_ref):
    e = pl.program_id(0)
    n = out_ref.shape[0]
    ch = 512
    mc = mc_ref[...]
    pc = pc_ref[...]
    iota_rw = jax.lax.broadcasted_iota(jnp.int32, (ch, capacity), 1)
    for c in range(n // ch):
        sl = slice(c * ch, (c + 1) * ch)
        e1c = mc[sl, 0:1]
        e2c = mc[sl, 1:2]
        r1c = mc[sl, 2:3]
        r2c = mc[sl, 3:4]
        p1c = pc[sl, 0:1]
        p2c = pc[sl, 1:2]
        w = (jnp.where((e1c == e) & (iota_rw == r1c), p1c, 0.0)
             + jnp.where((e2c == e) & (iota_rw == r2c), p2c, 0.0))
        w_ref[sl, :] = w.astype(jnp.bfloat16)
    dbf = d_ref[0]
    for c in range(n // ch):
        sl = slice(c * ch, (c + 1) * ch)
        contrib = jnp.dot(w_ref[sl, :], dbf, preferred_element_type=jnp.float32)

        @pl.when(e == 0)
        def _():
            out_ref[sl, :] = contrib

        @pl.when(e > 0)
        def _():
            out_ref[sl, :] = out_ref[sl, :] + contrib


def kernel(hidden_states, w_router, w_gate, w_up, w_down):
    n, hidden = hidden_states.shape
    num_experts = w_router.shape[1]
    ffn = w_gate.shape[2]
    capacity = max(4, math.ceil(1.25 * n * 2 / num_experts))
    capacity = min(capacity, n * 2)

    mc, mr, pc, aux = pl.pallas_call(
        functools.partial(_router_body, num_experts, capacity),
        in_specs=[
            pl.BlockSpec((n, hidden), lambda: (0, 0)),
            pl.BlockSpec((hidden, num_experts), lambda: (0, 0)),
        ],
        out_specs=[
            pl.BlockSpec((n, 8), lambda: (0, 0)),
            pl.BlockSpec((8, n), lambda: (0, 0)),
            pl.BlockSpec((n, 8), lambda: (0, 0)),
            pl.BlockSpec((1, 1), lambda: (0, 0)),
        ],
        out_shape=[
            jax.ShapeDtypeStruct((n, 8), jnp.int32),
            jax.ShapeDtypeStruct((8, n), jnp.int32),
            jax.ShapeDtypeStruct((n, 8), jnp.float32),
            jax.ShapeDtypeStruct((1, 1), jnp.float32),
        ],
    )(hidden_states, w_router)

    x_bf = hidden_states.astype(jnp.bfloat16)

    tile = 512
    nt = ffn // tile
    d_all = pl.pallas_call(
        functools.partial(_ffn_body, capacity, nt),
        grid=(num_experts, nt),
        in_specs=[
            pl.BlockSpec((n, hidden), lambda e, f: (0, 0)),
            pl.BlockSpec((1, hidden, tile), lambda e, f: (e, 0, f)),
            pl.BlockSpec((1, hidden, tile), lambda e, f: (e, 0, f)),
            pl.BlockSpec((1, tile, hidden), lambda e, f: (e, f, 0)),
            pl.BlockSpec((8, n), lambda e, f: (0, 0)),
        ],
        out_specs=pl.BlockSpec((1, capacity, hidden), lambda e, f: (e, 0, 0)),
        out_shape=jax.ShapeDtypeStruct((num_experts, capacity, hidden),
                                       jnp.bfloat16),
        scratch_shapes=[
            pltpu.VMEM((capacity, n), jnp.bfloat16),
            pltpu.VMEM((capacity, hidden), jnp.bfloat16),
        ],
    )(x_bf, w_gate, w_up, w_down, mr)

    out = pl.pallas_call(
        functools.partial(_combine_body, capacity),
        grid=(num_experts,),
        in_specs=[
            pl.BlockSpec((n, 8), lambda e: (0, 0)),
            pl.BlockSpec((n, 8), lambda e: (0, 0)),
            pl.BlockSpec((1, capacity, hidden), lambda e: (e, 0, 0)),
        ],
        out_specs=pl.BlockSpec((n, hidden), lambda e: (0, 0)),
        out_shape=jax.ShapeDtypeStruct((n, hidden), jnp.float32),
        scratch_shapes=[
            pltpu.VMEM((n, capacity), jnp.bfloat16),
        ],
    )(mc, pc, d_all)

    return out, aux[0, 0]


# speedup vs baseline: 1.5268x; 1.0263x over previous
import functools
import math

import jax
import jax.numpy as jnp
from jax.experimental import pallas as pl
from jax.experimental.pallas import tpu as pltpu


def _router_body(num_experts, capacity, x_ref, wr_ref, mc_ref, mr_ref, pc_ref,
                 aux_ref):
    n = x_ref.shape[0]
    e_dim = num_experts
    x = x_ref[...]
    wr = wr_ref[...]
    logits = jnp.dot(x, wr, preferred_element_type=jnp.float32)
    mx = jnp.max(logits, axis=-1, keepdims=True)
    ex = jnp.exp(logits - mx)
    probs = ex / jnp.sum(ex, axis=-1, keepdims=True)

    iota_e = jax.lax.broadcasted_iota(jnp.int32, (n, e_dim), 1)
    m1 = jnp.max(probs, axis=-1, keepdims=True)
    e1 = jnp.min(jnp.where(probs == m1, iota_e, e_dim), axis=-1, keepdims=True)
    probs2 = jnp.where(iota_e == e1, -1.0, probs)
    m2 = jnp.max(probs2, axis=-1, keepdims=True)
    e2 = jnp.min(jnp.where(probs2 == m2, iota_e, e_dim), axis=-1, keepdims=True)
    s = m1 + m2
    p1 = m1 / s
    p2 = m2 / s

    imp = jnp.sum(probs, axis=0, keepdims=True)
    load = jnp.sum((iota_e == e1).astype(jnp.float32)
                   + (iota_e == e2).astype(jnp.float32), axis=0, keepdims=True)
    aux_ref[...] = (e_dim * jnp.sum(imp * load, keepdims=True)) / (n * 2)

    e1f = e1.astype(jnp.float32)
    e2f = e2.astype(jnp.float32)
    zeros4 = jnp.zeros((n, 4), jnp.float32)
    packed = jnp.concatenate([e1f, e2f, p1, p2, zeros4], axis=1)
    packed_t = packed.T
    e1r = packed_t[0:1, :].astype(jnp.int32)
    e2r = packed_t[1:2, :].astype(jnp.int32)
    p1r = packed_t[2:3, :]
    p2r = packed_t[3:4, :]

    ch = 512
    tok_row = jax.lax.broadcasted_iota(jnp.int32, (1, n), 1)
    r1_parts = []
    r2_parts = []
    for c in range(n // ch):
        sl = slice(c * ch, (c + 1) * ch)
        tcol = jax.lax.broadcasted_iota(jnp.int32, (ch, 1), 0) + c * ch
        for k, (ei, pi) in enumerate(((e1[sl], p1[sl]), (e2[sl], p2[sl]))):
            cnt = jnp.zeros((ch, 1), jnp.int32)
            for kj, (ejr, pjr) in enumerate(((e1r, p1r), (e2r, p2r))):
                same = ejr == ei
                if kj < k:
                    earlier = tok_row <= tcol
                else:
                    earlier = tok_row < tcol
                beat = (pjr > pi) | ((pjr == pi) & earlier)
                cnt = cnt + jnp.sum((same & beat).astype(jnp.int32), axis=1,
                                    keepdims=True)
            if k == 0:
                r1_parts.append(cnt)
            else:
                r2_parts.append(cnt)
    r1 = jnp.concatenate(r1_parts, axis=0)
    r2 = jnp.concatenate(r2_parts, axis=0)

    zeros4i = jnp.zeros((n, 4), jnp.int32)
    mc_ref[...] = jnp.concatenate([e1, e2, r1, r2, zeros4i], axis=1)
    packed2 = jnp.concatenate([e1f, e2f, r1.astype(jnp.float32),
                               r2.astype(jnp.float32), zeros4], axis=1)
    mr_ref[...] = packed2.T.astype(jnp.int32)
    pc_ref[...] = jnp.concatenate([p1, p2, zeros4, jnp.zeros((n, 2),
                                                             jnp.float32)],
                                  axis=1)


def _ffn_body(capacity, nt, x_ref, wg_ref, wu_ref, wd_ref, mr_ref, d_ref,
              g_ref, seg_ref):
    e = pl.program_id(0)
    f = pl.program_id(1)
    n = x_ref.shape[0]
    ch = 512

    @pl.when(f == 0)
    def _build_seg():
        mr = mr_ref[...]
        for c in range(n // ch):
            sl = slice(c * ch, (c + 1) * ch)
            e1r = mr[0:1, sl]
            e2r = mr[1:2, sl]
            r1r = mr[2:3, sl]
            r2r = mr[3:4, sl]
            iota_r = jax.lax.broadcasted_iota(jnp.int32, (capacity, ch), 0)
            g_ref[:, sl] = (
                ((e1r == e) & (iota_r == r1r)).astype(jnp.bfloat16)
                + ((e2r == e) & (iota_r == r2r)).astype(jnp.bfloat16))
        seg32 = jnp.dot(g_ref[...], x_ref[...],
                        preferred_element_type=jnp.float32)
        seg_ref[...] = seg32.astype(jnp.bfloat16)

    gate = jnp.dot(seg_ref[...], wg_ref[0].astype(jnp.bfloat16),
                   preferred_element_type=jnp.float32)
    up = jnp.dot(seg_ref[...], wu_ref[0].astype(jnp.bfloat16),
                 preferred_element_type=jnp.float32)
    h = (gate * jax.nn.sigmoid(gate)) * up
    dpart = jnp.dot(h.astype(jnp.bfloat16), wd_ref[0].astype(jnp.bfloat16),
                    preferred_element_type=jnp.float32)

    @pl.when(f == 0)
    def _init_d():
        d_ref[0] = dpart.astype(jnp.bfloat16)

    @pl.when(f > 0)
    def _acc_d():
        d_ref[0] = (d_ref[0].astype(jnp.float32) + dpart).astype(jnp.bfloat16)


def _combine_body(capacity, num_experts, mc_ref, pc_ref, d_ref, out_ref,
                  w_ref):
    ch = out_ref.shape[0]
    hidden = out_ref.shape[1]
    mc = mc_ref[...]
    pc = pc_ref[...]
    e1c = mc[:, 0:1]
    e2c = mc[:, 1:2]
    r1c = mc[:, 2:3]
    r2c = mc[:, 3:4]
    p1c = pc[:, 0:1]
    p2c = pc[:, 1:2]
    iota_rw = jax.lax.broadcasted_iota(jnp.int32, (ch, capacity), 1)
    acc = jnp.zeros((ch, hidden), jnp.float32)
    for e in range(num_experts):
        w = (jnp.where((e1c == e) & (iota_rw == r1c), p1c, 0.0)
             + jnp.where((e2c == e) & (iota_rw == r2c), p2c, 0.0))
        w_ref[...] = w.astype(jnp.bfloat16)
        acc = acc + jnp.dot(w_ref[...], d_ref[e],
                            preferred_element_type=jnp.float32)
    out_ref[...] = acc


def kernel(hidden_states, w_router, w_gate, w_up, w_down):
    n, hidden = hidden_states.shape
    num_experts = w_router.shape[1]
    ffn = w_gate.shape[2]
    capacity = max(4, math.ceil(1.25 * n * 2 / num_experts))
    capacity = min(capacity, n * 2)

    mc, mr, pc, aux = pl.pallas_call(
        functools.partial(_router_body, num_experts, capacity),
        in_specs=[
            pl.BlockSpec((n, hidden), lambda: (0, 0)),
            pl.BlockSpec((hidden, num_experts), lambda: (0, 0)),
        ],
        out_specs=[
            pl.BlockSpec((n, 8), lambda: (0, 0)),
            pl.BlockSpec((8, n), lambda: (0, 0)),
            pl.BlockSpec((n, 8), lambda: (0, 0)),
            pl.BlockSpec((1, 1), lambda: (0, 0)),
        ],
        out_shape=[
            jax.ShapeDtypeStruct((n, 8), jnp.int32),
            jax.ShapeDtypeStruct((8, n), jnp.int32),
            jax.ShapeDtypeStruct((n, 8), jnp.float32),
            jax.ShapeDtypeStruct((1, 1), jnp.float32),
        ],
    )(hidden_states, w_router)

    x_bf = hidden_states.astype(jnp.bfloat16)

    tile = 512
    nt = ffn // tile
    d_all = pl.pallas_call(
        functools.partial(_ffn_body, capacity, nt),
        grid=(num_experts, nt),
        in_specs=[
            pl.BlockSpec((n, hidden), lambda e, f: (0, 0)),
            pl.BlockSpec((1, hidden, tile), lambda e, f: (e, 0, f)),
            pl.BlockSpec((1, hidden, tile), lambda e, f: (e, 0, f)),
            pl.BlockSpec((1, tile, hidden), lambda e, f: (e, f, 0)),
            pl.BlockSpec((8, n), lambda e, f: (0, 0)),
        ],
        out_specs=pl.BlockSpec((1, capacity, hidden), lambda e, f: (e, 0, 0)),
        out_shape=jax.ShapeDtypeStruct((num_experts, capacity, hidden),
                                       jnp.bfloat16),
        scratch_shapes=[
            pltpu.VMEM((capacity, n), jnp.bfloat16),
            pltpu.VMEM((capacity, hidden), jnp.bfloat16),
        ],
    )(x_bf, w_gate, w_up, w_down, mr)

    cch = 512
    out = pl.pallas_call(
        functools.partial(_combine_body, capacity, num_experts),
        grid=(n // cch,),
        in_specs=[
            pl.BlockSpec((cch, 8), lambda c: (c, 0)),
            pl.BlockSpec((cch, 8), lambda c: (c, 0)),
            pl.BlockSpec((num_experts, capacity, hidden), lambda c: (0, 0, 0)),
        ],
        out_specs=pl.BlockSpec((cch, hidden), lambda c: (c, 0)),
        out_shape=jax.ShapeDtypeStruct((n, hidden), jnp.float32),
        scratch_shapes=[
            pltpu.VMEM((cch, capacity), jnp.bfloat16),
        ],
    )(mc, pc, d_all)

    return out, aux[0, 0]
